# baseline (speedup 1.0000x reference)
import functools

import jax
import jax.numpy as jnp
from jax import lax
from jax.experimental import pallas as pl
from jax.experimental.pallas import tpu as pltpu
from jax.experimental.pallas import tpu_sc as plsc

_B = 16384
_NC, _NS = 2, 16
_NW = _NC * _NS
_BPW = _B // _NW
_CH = 128
_NCH = _BPW // _CH
_RING = 4

_HIDDEN = 128
_NTAB = 4


def _sc_gather(idxs, tabs):
    mesh = plsc.VectorSubcoreMesh(core_axis_name="c", subcore_axis_name="s")

    @functools.partial(
        pl.kernel,
        mesh=mesh,
        out_type=[jax.ShapeDtypeStruct((_B, 128), jnp.float32)
                  for _ in range(_NTAB)],
        scratch_types=[
            pltpu.VMEM((_NCH, _CH), jnp.int32),
            pltpu.VMEM((_NCH, _CH), jnp.int32),
            pltpu.VMEM((_NCH, _CH), jnp.int32),
            pltpu.VMEM((_NCH, _CH), jnp.int32),
            pltpu.VMEM((_RING, _CH, 128), jnp.float32),
            pltpu.SemaphoreType.DMA((_RING,)),
            pltpu.SemaphoreType.DMA((_RING,)),
            pltpu.SemaphoreType.DMA,
        ],
    )
    def k(i0, i1, i2, i3, t0, t1, t2, t3,
          o0, o1, o2, o3, x0, x1, x2, x3, ring, gsem, wsem, isem):
        wid = lax.axis_index("s") * _NC + lax.axis_index("c")
        base = wid * _BPW
        row = wid * _NCH
        idx_loads = [
            pltpu.async_copy(i0.at[pl.ds(row, _NCH)], x0, isem),
            pltpu.async_copy(i1.at[pl.ds(row, _NCH)], x1, isem),
            pltpu.async_copy(i2.at[pl.ds(row, _NCH)], x2, isem),
            pltpu.async_copy(i3.at[pl.ds(row, _NCH)], x3, isem),
        ]
        for c in idx_loads:
            c.wait()

        tasks = [(x0, t0, o0), (x1, t1, o1), (x2, t2, o2), (x3, t3, o3)]
        flat = [(x, t, o, j) for (x, t, o) in tasks for j in range(_NCH)]
        n = len(flat)

        def gather(k_, s):
            x, t, _, j = flat[k_]
            return pltpu.async_copy(t.at[x.at[j]], ring.at[s], gsem.at[s])

        def write(k_, s):
            _, _, o, j = flat[k_]
            dst = o.at[pl.ds(base + j * _CH, _CH)]
            return pltpu.async_copy(ring.at[s], dst, wsem.at[s])

        started_g = [gather(k_, k_ % _RING) for k_ in range(_RING)]
        started_w = []
        for k_ in range(n):
            s = k_ % _RING
            started_g[s].wait()
            started_w.append(write(k_, s))
            nk = k_ + _RING
            if nk < n:
                started_w[k_].wait()
                started_g[s] = gather(nk, s)
        for k_ in range(n - _RING, n):
            started_w[k_].wait()

    return k(*idxs, *tabs)


_BLK = 2048


def _extract(buf, off, width):
    acc = buf[:, 0:width]
    for k in range(1, 128 // width):
        acc = jnp.where(off == k, buf[:, k * width:(k + 1) * width], acc)
    return acc


def _mlp_body(t_ref, dy_ref, nm_ref, do_ref, ao_ref, uo_ref, io_ref,
              d_ref, a_ref, u_ref, i_ref,
              tt_ref, dt_ref, w1_ref, b1_ref, w2_ref, b2_ref, o_ref):
    t_oh = (lax.broadcasted_iota(jnp.int32, (_BLK, 8), 1) == t_ref[...]
            ).astype(jnp.float32)
    dy_oh = (lax.broadcasted_iota(jnp.int32, (_BLK, 8), 1) == dy_ref[...]
             ).astype(jnp.float32)
    type_emb = jnp.dot(t_oh, tt_ref[...], preferred_element_type=jnp.float32)
    day_emb = jnp.dot(dy_oh, dt_ref[...], preferred_element_type=jnp.float32)
    dom = _extract(d_ref[...], do_ref[...], 16)
    auth = _extract(a_ref[...], ao_ref[...], 16)
    user = _extract(u_ref[...], uo_ref[...], 16)
    item = _extract(i_ref[...], io_ref[...], 8)
    x = jnp.concatenate(
        [type_emb, day_emb, dom, auth, user, item, nm_ref[...]], axis=1)
    h = jnp.maximum(
        jnp.dot(x, w1_ref[...], preferred_element_type=jnp.float32)
        + b1_ref[...], 0.0)
    o_ref[...] = (jnp.dot(h, w2_ref[...], preferred_element_type=jnp.float32)
                  + b2_ref[...])


def _tc_mlp(t2, d2, num3, offs, dom, auth, user, item, type_t, day_t8,
            W1, b1, W2, b2):
    grid = (_B // _BLK,)
    full = lambda shape: pl.BlockSpec(shape, lambda i: (0, 0))
    blk = lambda w: pl.BlockSpec((_BLK, w), lambda i: (i, 0))
    return pl.pallas_call(
        _mlp_body,
        grid=grid,
        in_specs=[
            blk(1), blk(1), blk(3),
            blk(1), blk(1), blk(1), blk(1),
            blk(128), blk(128), blk(128), blk(128),
            full((8, 8)), full((8, 4)),
            full((71, _HIDDEN)), full((1, _HIDDEN)),
            full((_HIDDEN, 1)), full((1, 1)),
        ],
        out_specs=blk(1),
        out_shape=jax.ShapeDtypeStruct((_B, 1), jnp.float32),
    )(t2, d2, num3, *offs, dom, auth, user, item, type_t, day_t8,
      W1, b1, W2, b2)


def kernel(type_id, day_of_week_id, domain_id, author_id, user_id, item_id,
           hour_of_day, karma, descendants,
           type_table, day_table, domain_table, author_table, user_table,
           item_table, W1, b1, W2, b2):
    dom_id = domain_id.astype(jnp.int32)
    auth_id = author_id.astype(jnp.int32)
    usr_id = user_id.astype(jnp.int32)
    itm_id = item_id.astype(jnp.int32)

    idxs = [
        (dom_id >> 3).reshape(_B // _CH, _CH),
        (auth_id >> 3).reshape(_B // _CH, _CH),
        (usr_id >> 3).reshape(_B // _CH, _CH),
        (itm_id >> 4).reshape(_B // _CH, _CH),
    ]
    tabs = [
        domain_table.reshape(-1, 128),
        author_table.reshape(-1, 128),
        user_table.reshape(-1, 128),
        item_table.reshape(-1, 128),
    ]
    dom, auth, user, item = _sc_gather(idxs, tabs)

    offs = [
        (dom_id & 7).reshape(_B, 1),
        (auth_id & 7).reshape(_B, 1),
        (usr_id & 7).reshape(_B, 1),
        (itm_id & 15).reshape(_B, 1),
    ]
    num3 = jnp.stack([hour_of_day.astype(jnp.float32),
                      karma.astype(jnp.float32),
                      descendants.astype(jnp.float32)], axis=1)
    t2 = type_id.astype(jnp.int32).reshape(_B, 1)
    d2 = day_of_week_id.astype(jnp.int32).reshape(_B, 1)
    day_t8 = jnp.zeros((8, 4), day_table.dtype).at[:7].set(day_table)
    out = _tc_mlp(t2, d2, num3, offs, dom, auth, user, item,
                  type_table, day_t8,
                  W1, b1.reshape(1, _HIDDEN), W2, b2.reshape(1, 1))
    return out.reshape(_B)


# speedup vs baseline: 7.1963x; 7.1963x over previous
import functools

import jax
import jax.numpy as jnp
from jax import lax
from jax.experimental import pallas as pl
from jax.experimental.pallas import tpu as pltpu
from jax.experimental.pallas import tpu_sc as plsc

_B = 16384
_NC, _NS = 2, 16
_NW = _NC * _NS
_BPW = _B // _NW
_CH = 128
_NCH = _BPW // _CH
_RING = 4

_HIDDEN = 128
_NTAB = 4


_CHUNK = 65536


def _pack_table(tab_t, d, chunk=_CHUNK):
    v = tab_t.shape[1]
    per = 128 // d
    rpb = chunk // per
    nblk = (v + chunk - 1) // chunk
    grid = (nblk,)

    def body(x_ref, o_ref):
        x = x_ref[...].astype(jnp.bfloat16)
        xs = jnp.concatenate(
            [x[:, rpb * j:rpb * (j + 1)] for j in range(per)], axis=0)
        eye = (lax.broadcasted_iota(jnp.int32, (128, 128), 0)
               == lax.broadcasted_iota(jnp.int32, (128, 128), 1)
               ).astype(jnp.bfloat16)
        o_ref[...] = lax.dot_general(
            xs, eye, (((0,), (0,)), ((), ())),
            preferred_element_type=jnp.float32)

    return pl.pallas_call(
        body,
        grid=grid,
        in_specs=[pl.BlockSpec((d, chunk), lambda i: (0, i))],
        out_specs=pl.BlockSpec((rpb, 128), lambda i: (i, 0)),
        out_shape=jax.ShapeDtypeStruct((nblk * rpb, 128), jnp.float32),
    )(tab_t)


def _sc_gather(idxs, tabs):
    mesh = plsc.VectorSubcoreMesh(core_axis_name="c", subcore_axis_name="s")

    @functools.partial(
        pl.kernel,
        mesh=mesh,
        out_type=[jax.ShapeDtypeStruct((_B, 128), jnp.float32)
                  for _ in range(_NTAB)],
        scratch_types=[
            pltpu.VMEM((_NCH, _CH), jnp.int32),
            pltpu.VMEM((_NCH, _CH), jnp.int32),
            pltpu.VMEM((_NCH, _CH), jnp.int32),
            pltpu.VMEM((_NCH, _CH), jnp.int32),
            pltpu.VMEM((_RING, _CH, 128), jnp.float32),
            pltpu.SemaphoreType.DMA((_RING,)),
            pltpu.SemaphoreType.DMA((_RING,)),
            pltpu.SemaphoreType.DMA,
        ],
        compiler_params=pltpu.CompilerParams(use_tc_tiling_on_sc=True),
    )
    def k(i0, i1, i2, i3, t0, t1, t2, t3,
          o0, o1, o2, o3, x0, x1, x2, x3, ring, gsem, wsem, isem):
        wid = lax.axis_index("s") * _NC + lax.axis_index("c")
        base = wid * _BPW
        row = wid * _NCH
        idx_loads = [
            pltpu.async_copy(i0.at[pl.ds(row, _NCH)], x0, isem),
            pltpu.async_copy(i1.at[pl.ds(row, _NCH)], x1, isem),
            pltpu.async_copy(i2.at[pl.ds(row, _NCH)], x2, isem),
            pltpu.async_copy(i3.at[pl.ds(row, _NCH)], x3, isem),
        ]
        for c in idx_loads:
            c.wait()

        tasks = [(x0, t0, o0), (x1, t1, o1), (x2, t2, o2), (x3, t3, o3)]
        flat = [(x, t, o, j) for (x, t, o) in tasks for j in range(_NCH)]
        n = len(flat)

        def gather(k_, s):
            x, t, _, j = flat[k_]
            return pltpu.async_copy(t.at[x.at[j]], ring.at[s], gsem.at[s])

        def write(k_, s):
            _, _, o, j = flat[k_]
            dst = o.at[pl.ds(base + j * _CH, _CH)]
            return pltpu.async_copy(ring.at[s], dst, wsem.at[s])

        started_g = [gather(k_, k_ % _RING) for k_ in range(_RING)]
        started_w = []
        for k_ in range(n):
            s = k_ % _RING
            started_g[s].wait()
            started_w.append(write(k_, s))
            nk = k_ + _RING
            if nk < n:
                started_w[k_].wait()
                started_g[s] = gather(nk, s)
        for k_ in range(n - _RING, n):
            started_w[k_].wait()

    return k(*idxs, *tabs)


_BLK = 2048


def _mask_slot(buf, off, width):
    lane = (lax.broadcasted_iota(jnp.int32, (_BLK, 128), 1)
            // width).astype(jnp.float32)
    return buf * (lane == off).astype(jnp.float32)


def _mlp_body(s_ref, d_ref, a_ref, u_ref, i_ref,
              w1big_ref, pt_ref, pd_ref, w1n_ref,
              b1_ref, w2_ref, b2_ref, o_ref):
    scal = s_ref[...]
    iota8 = lax.broadcasted_iota(jnp.int32, (_BLK, 8), 1).astype(jnp.float32)
    t_oh = (iota8 == scal[:, 0:1]).astype(jnp.bfloat16)
    dy_oh = (iota8 == scal[:, 1:2]).astype(jnp.bfloat16)
    xbig = jnp.concatenate(
        [_mask_slot(d_ref[...], scal[:, 2:3], 16),
         _mask_slot(a_ref[...], scal[:, 3:4], 16),
         _mask_slot(u_ref[...], scal[:, 4:5], 16),
         _mask_slot(i_ref[...], scal[:, 5:6], 8)],
        axis=1).astype(jnp.bfloat16)
    h = (jnp.dot(xbig, w1big_ref[...], preferred_element_type=jnp.float32)
         + jnp.dot(t_oh, pt_ref[...], preferred_element_type=jnp.float32)
         + jnp.dot(dy_oh, pd_ref[...], preferred_element_type=jnp.float32)
         + jnp.dot(scal[:, 6:9].astype(jnp.bfloat16), w1n_ref[...],
                   preferred_element_type=jnp.float32)
         + b1_ref[...])
    h = jnp.maximum(h, 0.0)
    out = lax.dot_general(w2_ref[...], h, (((1,), (1,)), ((), ())),
                          preferred_element_type=jnp.float32)
    o_ref[...] = out + b2_ref[...]


def _tc_mlp(scal, dom, auth, user, item, w1big, pt, pd, w1n, b1, w2t, b2):
    grid = (_B // _BLK,)
    full = lambda shape: pl.BlockSpec(shape, lambda i: (0, 0))
    blk = lambda w: pl.BlockSpec((_BLK, w), lambda i: (i, 0))
    return pl.pallas_call(
        _mlp_body,
        grid=grid,
        in_specs=[
            blk(9),
            blk(128), blk(128), blk(128), blk(128),
            full((512, _HIDDEN)), full((8, _HIDDEN)), full((8, _HIDDEN)),
            full((3, _HIDDEN)),
            full((1, _HIDDEN)), full((1, _HIDDEN)), full((1, 1)),
        ],
        out_specs=pl.BlockSpec((1, _BLK), lambda i: (0, i)),
        out_shape=jax.ShapeDtypeStruct((1, _B), jnp.float32),
    )(scal, dom, auth, user, item, w1big, pt, pd, w1n, b1, w2t, b2)


def kernel(type_id, day_of_week_id, domain_id, author_id, user_id, item_id,
           hour_of_day, karma, descendants,
           type_table, day_table, domain_table, author_table, user_table,
           item_table, W1, b1, W2, b2):
    dom_id = domain_id.astype(jnp.int32)
    auth_id = author_id.astype(jnp.int32)
    usr_id = user_id.astype(jnp.int32)
    itm_id = item_id.astype(jnp.int32)

    def _pidx(r, rpb):
        return (r >> 16) * rpb + (r & (rpb - 1))

    idxs = [
        _pidx(dom_id, 8192).reshape(_B // _CH, _CH),
        _pidx(auth_id, 8192).reshape(_B // _CH, _CH),
        _pidx(usr_id, 8192).reshape(_B // _CH, _CH),
        _pidx(itm_id, 4096).reshape(_B // _CH, _CH),
    ]
    tabs = [
        _pack_table(domain_table.T, 16),
        _pack_table(author_table.T, 16),
        _pack_table(user_table.T, 16),
        _pack_table(item_table.T, 8),
    ]
    dom, auth, user, item = _sc_gather(idxs, tabs)

    scal = jnp.stack(
        [type_id.astype(jnp.float32),
         day_of_week_id.astype(jnp.float32),
         ((dom_id >> 13) & 7).astype(jnp.float32),
         ((auth_id >> 13) & 7).astype(jnp.float32),
         ((usr_id >> 13) & 7).astype(jnp.float32),
         ((itm_id >> 12) & 15).astype(jnp.float32),
         hour_of_day.astype(jnp.float32),
         karma.astype(jnp.float32),
         descendants.astype(jnp.float32)], axis=1)
    day_t8 = jnp.zeros((8, 4), day_table.dtype).at[:7].set(day_table)
    w1big = jnp.concatenate(
        [jnp.tile(W1[12:28], (8, 1)), jnp.tile(W1[28:44], (8, 1)),
         jnp.tile(W1[44:60], (8, 1)), jnp.tile(W1[60:68], (16, 1))],
        axis=0).astype(jnp.bfloat16)
    pt = (type_table @ W1[0:8]).astype(jnp.bfloat16)
    pd = (day_t8 @ W1[8:12]).astype(jnp.bfloat16)
    w1n = W1[68:71].astype(jnp.bfloat16)
    out = _tc_mlp(scal, dom, auth, user, item,
                  w1big, pt, pd, w1n,
                  b1.reshape(1, _HIDDEN), W2.reshape(1, _HIDDEN),
                  b2.reshape(1, 1))
    return out.reshape(_B)
